# Initial kernel scaffold; baseline (speedup 1.0000x reference)
import jax
import jax.numpy as jnp
from jax.experimental import pallas as pl


def kernel(features, edge_index, edge_types, W_et, b_et, W_ih, W_hh, b_ih, b_hh, W_cls, b_cls):
    raise NotImplementedError("write your pallas kernel here")


# speedup vs baseline: 17.0221x; 17.0221x over previous
import functools

import jax
import jax.numpy as jnp
from jax import lax
from jax.experimental import pallas as pl
from jax.experimental.pallas import tpu as pltpu
from jax.experimental.pallas import tpu_sc as plsc

N = 10000
D = 128
T = 4
STEPS = 8
B = 10
NPG = 1000

NC = 2
NS = 16
NW = NC * NS
CH = 128
NCHUNK = 79
E_PAD = NW * CH * NCHUNK
NPAD = 10112
RPT = NPAD // NS

BN = 2000


def _transform_block(h_ref, w_ref, b_ref, out_ref):
    out_ref[0] = lax.dot_general(
        h_ref[...], w_ref[0], (((1,), (1,)), ((), ())),
        preferred_element_type=jnp.float32) + b_ref[0]


def _transform(h, W_et, b_et3):
    return pl.pallas_call(
        _transform_block,
        grid=(T, N // BN),
        in_specs=[
            pl.BlockSpec((BN, D), lambda t, i: (i, 0)),
            pl.BlockSpec((1, D, D), lambda t, i: (t, 0, 0)),
            pl.BlockSpec((1, 1, D), lambda t, i: (t, 0, 0)),
        ],
        out_specs=pl.BlockSpec((1, BN, D), lambda t, i: (t, i, 0)),
        out_shape=jax.ShapeDtypeStruct((T, N, D), jnp.float32),
    )(h, W_et, b_et3)


@functools.partial(
    pl.kernel,
    mesh=plsc.VectorSubcoreMesh(core_axis_name="c", subcore_axis_name="s"),
    out_type=jax.ShapeDtypeStruct((NC, NPAD, D), jnp.float32),
    scratch_types=[
        pltpu.VMEM((NCHUNK, CH), jnp.int32),
        pltpu.VMEM((NCHUNK, CH), jnp.int32),
        pltpu.VMEM((CH, D), jnp.float32),
        pltpu.VMEM_SHARED((NPAD, D), jnp.float32),
        pltpu.SemaphoreType.DMA,
    ],
)
def _sc_aggregate(table, idxs, dsts, zrows, out, idx_blk, dst_blk, rows, acc, sem):
    c = lax.axis_index("c")
    s = lax.axis_index("s")
    wid = c * NS + s
    pltpu.sync_copy(zrows.at[pl.ds(s * RPT, RPT)], acc.at[pl.ds(s * RPT, RPT)])
    pltpu.sync_copy(idxs.at[wid], idx_blk)
    pltpu.sync_copy(dsts.at[wid], dst_blk)
    plsc.subcore_barrier()

    def chunk(j, carry):
        pltpu.async_copy(table.at[idx_blk.at[j]], rows, sem).wait()
        pltpu.sync_copy(rows, acc.at[dst_blk.at[j]], add=True)
        return carry

    lax.fori_loop(0, NCHUNK, chunk, 0)
    plsc.subcore_barrier()
    pltpu.sync_copy(acc.at[pl.ds(s * RPT, RPT)], out.at[c, pl.ds(s * RPT, RPT)])


def _gru_block(p0_ref, p1_ref, h_ref, wih_ref, whh_ref, bih_ref, bhh_ref, out_ref):
    a = p0_ref[...] + p1_ref[...]
    h = h_ref[...]
    gi = lax.dot_general(a, wih_ref[...], (((1,), (1,)), ((), ())),
                         preferred_element_type=jnp.float32) + bih_ref[...]
    gh = lax.dot_general(h, whh_ref[...], (((1,), (1,)), ((), ())),
                         preferred_element_type=jnp.float32) + bhh_ref[...]
    r = jax.nn.sigmoid(gi[:, :D] + gh[:, :D])
    z = jax.nn.sigmoid(gi[:, D:2 * D] + gh[:, D:2 * D])
    n = jnp.tanh(gi[:, 2 * D:] + r * gh[:, 2 * D:])
    out_ref[...] = (1.0 - z) * n + z * h


def _gru(p0, p1, h, W_ih, W_hh, bih2, bhh2):
    return pl.pallas_call(
        _gru_block,
        grid=(N // BN,),
        in_specs=[
            pl.BlockSpec((BN, D), lambda i: (i, 0)),
            pl.BlockSpec((BN, D), lambda i: (i, 0)),
            pl.BlockSpec((BN, D), lambda i: (i, 0)),
            pl.BlockSpec((3 * D, D), lambda i: (0, 0)),
            pl.BlockSpec((3 * D, D), lambda i: (0, 0)),
            pl.BlockSpec((1, 3 * D), lambda i: (0, 0)),
            pl.BlockSpec((1, 3 * D), lambda i: (0, 0)),
        ],
        out_specs=pl.BlockSpec((BN, D), lambda i: (i, 0)),
        out_shape=jax.ShapeDtypeStruct((N, D), jnp.float32),
    )(p0, p1, h, W_ih, W_hh, bih2, bhh2)


def _cls_block(h_ref, w_ref, b_ref, out_ref):
    pooled = h_ref[...].reshape(B, NPG, D).sum(axis=1)
    prod = pooled * w_ref[...]
    ssum = lax.dot_general(prod, jnp.ones((D, D), jnp.float32),
                           (((1,), (0,)), ((), ())),
                           preferred_element_type=jnp.float32)
    out_ref[...] = jax.nn.sigmoid(ssum + b_ref[0, 0])


def _cls(h, W_cls, b_cls2):
    return pl.pallas_call(
        _cls_block,
        out_shape=jax.ShapeDtypeStruct((B, D), jnp.float32),
    )(h, W_cls, b_cls2)


def kernel(features, edge_index, edge_types, W_et, b_et, W_ih, W_hh, b_ih, b_hh,
           W_cls, b_cls):
    src = edge_index[0]
    dst = edge_index[1]
    e = src.shape[0]
    pad = E_PAD - e
    gidx = (edge_types * N + src).astype(jnp.int32)
    gidx = jnp.concatenate([gidx, jnp.zeros((pad,), jnp.int32)]).reshape(
        NW, NCHUNK, CH)
    dsts = jnp.concatenate([dst, jnp.full((pad,), N, jnp.int32)]).reshape(
        NW, NCHUNK, CH)
    zrows = jnp.zeros((NPAD, D), jnp.float32)
    bih2 = b_ih.reshape(1, 3 * D)
    bhh2 = b_hh.reshape(1, 3 * D)
    b_et3 = b_et.reshape(T, 1, D)

    h = features
    for _ in range(STEPS):
        table = _transform(h, W_et, b_et3).reshape(T * N, D)
        parts = _sc_aggregate(table, gidx, dsts, zrows)
        h = _gru(parts[0, :N], parts[1, :N], h, W_ih, W_hh, bih2, bhh2)
    out2 = _cls(h, W_cls, b_cls.reshape(1, 1))
    return out2[:, 0]
